# Initial kernel scaffold; baseline (speedup 1.0000x reference)
import jax
import jax.numpy as jnp
from jax.experimental import pallas as pl


def kernel(x, edge_index, edge_weight, W0, b0, W1, b1):
    raise NotImplementedError("write your pallas kernel here")


# speedup vs baseline: 18.3153x; 18.3153x over previous
import functools

import jax
import jax.numpy as jnp
from jax import lax
from jax.experimental import pallas as pl
from jax.experimental.pallas import tpu as pltpu
from jax.experimental.pallas import tpu_sc as plsc

NC = 2
NS = 16
NW = NC * NS
K = 80


def _sc_mesh():
    return plsc.VectorSubcoreMesh(
        core_axis_name="c", subcore_axis_name="s",
        num_cores=NC, num_subcores=NS)


def _deg_sc(row3, w3, zeros_n, npad, chunks):
    rpt = npad // NS

    @functools.partial(
        pl.kernel,
        out_type=jax.ShapeDtypeStruct((NC * npad,), jnp.float32),
        mesh=_sc_mesh(),
        scratch_types=[
            pltpu.VMEM((chunks, K), jnp.int32),
            pltpu.VMEM((chunks, K), jnp.float32),
            pltpu.VMEM((rpt,), jnp.float32),
            pltpu.VMEM_SHARED((npad,), jnp.float32),
        ],
        compiler_params=pltpu.CompilerParams(use_tc_tiling_on_sc=False, needs_layout_passes=False),
    )
    def deg(row_hbm, w_hbm, z_hbm, out_hbm, row_t, w_t, obuf, acc):
        c = lax.axis_index("c")
        s = lax.axis_index("s")
        wid = s * NC + c
        pltpu.sync_copy(z_hbm.at[pl.ds(s * rpt, rpt)], obuf)
        pltpu.sync_copy(obuf, acc.at[pl.ds(s * rpt, rpt)])
        pltpu.sync_copy(row_hbm.at[wid], row_t)
        pltpu.sync_copy(w_hbm.at[wid], w_t)
        plsc.subcore_barrier()

        def chunk_body(i, carry):
            pltpu.sync_copy(w_t.at[i], acc.at[row_t.at[i]], add=True)
            return carry

        lax.fori_loop(0, chunks, chunk_body, 0)
        plsc.subcore_barrier()
        pltpu.sync_copy(acc.at[pl.ds(s * rpt, rpt)], obuf)
        pltpu.sync_copy(obuf, out_hbm.at[pl.ds(c * npad + s * rpt, rpt)])

    return deg(row3, w3, zeros_n)


def _spmm_sc(table, col3, row3, w3, zeros_nf, npad, f, chunks):
    rpt = npad // NS

    @functools.partial(
        pl.kernel,
        out_type=jax.ShapeDtypeStruct((NC, npad, f), jnp.float32),
        mesh=_sc_mesh(),
        scratch_types=[
            pltpu.VMEM((chunks, K), jnp.int32),
            pltpu.VMEM((chunks, K), jnp.int32),
            pltpu.VMEM((chunks, K), jnp.float32),
            pltpu.VMEM((K, f), jnp.float32),
            pltpu.VMEM((rpt, f), jnp.float32),
            pltpu.VMEM_SHARED((npad, f), jnp.float32),
            pltpu.SemaphoreType.DMA,
        ],
        compiler_params=pltpu.CompilerParams(use_tc_tiling_on_sc=False, needs_layout_passes=False),
    )
    def spmm(table_hbm, col_hbm, row_hbm, w_hbm, z_hbm, out_hbm,
             col_t, row_t, w_t, gbuf, obuf, acc, sem):
        c = lax.axis_index("c")
        s = lax.axis_index("s")
        wid = s * NC + c
        pltpu.sync_copy(z_hbm.at[pl.ds(s * rpt, rpt)], obuf)
        pltpu.sync_copy(obuf, acc.at[pl.ds(s * rpt, rpt)])
        pltpu.sync_copy(col_hbm.at[wid], col_t)
        pltpu.sync_copy(row_hbm.at[wid], row_t)
        pltpu.sync_copy(w_hbm.at[wid], w_t)
        plsc.subcore_barrier()

        def chunk_body(i, carry):
            pltpu.async_copy(table_hbm.at[col_t.at[i]], gbuf, sem).wait()
            ivec = jnp.full((16,), i, jnp.int32)
            for e in range(K):
                wb = plsc.load_gather(
                    w_t, [ivec, jnp.full((16,), e, jnp.int32)])
                for j in range(f // 16):
                    sl = pl.ds(j * 16, 16)
                    gbuf[e, sl] = gbuf[e, sl] * wb
            pltpu.sync_copy(gbuf, acc.at[row_t.at[i]], add=True)
            return carry

        lax.fori_loop(0, chunks, chunk_body, 0)
        plsc.subcore_barrier()
        pltpu.sync_copy(acc.at[pl.ds(s * rpt, rpt)], obuf)
        pltpu.sync_copy(obuf, out_hbm.at[c, pl.ds(s * rpt, rpt)])

    return spmm(table, col3, row3, w3, zeros_nf)


def _tc1(x, W0, degp, n, d, h, b):
    def body(x_ref, w0_ref, degp_ref, g0_ref, dinv_ref):
        deg = degp_ref[:, 0:1] + degp_ref[:, 1:2] + 1.0
        dinv = lax.rsqrt(deg)
        hh = jnp.dot(x_ref[...], w0_ref[...],
                     preferred_element_type=jnp.float32)
        g0_ref[...] = hh * dinv
        dinv_ref[...] = dinv

    grid = n // b
    return pl.pallas_call(
        body,
        grid=(grid,),
        in_specs=[
            pl.BlockSpec((b, d), lambda i: (i, 0)),
            pl.BlockSpec((d, h), lambda i: (0, 0)),
            pl.BlockSpec((b, NC), lambda i: (i, 0)),
        ],
        out_specs=[
            pl.BlockSpec((b, h), lambda i: (i, 0)),
            pl.BlockSpec((b, 1), lambda i: (i, 0)),
        ],
        out_shape=[
            jax.ShapeDtypeStruct((n, h), jnp.float32),
            jax.ShapeDtypeStruct((n, 1), jnp.float32),
        ],
    )(x, W0, degp)


def _tc2(p1, g0, dinv, b0r, W1p, n, h, cp, b):
    def body(p_ref, g0_ref, dinv_ref, b0_ref, w1_ref, g1_ref):
        ssum = p_ref[0] + p_ref[1] + g0_ref[...]
        a = jnp.maximum(dinv_ref[...] * ssum + b0_ref[...], 0.0)
        g1_ref[...] = jnp.dot(a, w1_ref[...],
                              preferred_element_type=jnp.float32) * dinv_ref[...]

    grid = n // b
    return pl.pallas_call(
        body,
        grid=(grid,),
        in_specs=[
            pl.BlockSpec((NC, b, h), lambda i: (0, i, 0)),
            pl.BlockSpec((b, h), lambda i: (i, 0)),
            pl.BlockSpec((b, 1), lambda i: (i, 0)),
            pl.BlockSpec((1, h), lambda i: (0, 0)),
            pl.BlockSpec((h, cp), lambda i: (0, 0)),
        ],
        out_specs=pl.BlockSpec((b, cp), lambda i: (i, 0)),
        out_shape=jax.ShapeDtypeStruct((n, cp), jnp.float32),
    )(p1, g0, dinv, b0r, W1p)


def _tc3(p2, g1, dinv, b1r, n, cp, b):
    def body(p_ref, g1_ref, dinv_ref, b1_ref, o_ref):
        ssum = p_ref[0] + p_ref[1] + g1_ref[...]
        o_ref[...] = dinv_ref[...] * ssum + b1_ref[...]

    grid = n // b
    return pl.pallas_call(
        body,
        grid=(grid,),
        in_specs=[
            pl.BlockSpec((NC, b, cp), lambda i: (0, i, 0)),
            pl.BlockSpec((b, cp), lambda i: (i, 0)),
            pl.BlockSpec((b, 1), lambda i: (i, 0)),
            pl.BlockSpec((1, cp), lambda i: (0, 0)),
        ],
        out_specs=pl.BlockSpec((b, cp), lambda i: (i, 0)),
        out_shape=jax.ShapeDtypeStruct((n, cp), jnp.float32),
    )(p2, g1, dinv, b1r)


def kernel(x, edge_index, edge_weight, W0, b0, W1, b1):
    n, d = x.shape
    h = W0.shape[1]
    c_out = W1.shape[1]
    cp = 48
    e = edge_index.shape[1]
    chunks = e // (NW * K)
    assert e == NW * chunks * K and n % NS == 0

    npad = 10240
    assert npad % (8 * NS) == 0 and npad >= n

    row = edge_index[0]
    col = edge_index[1]
    row3 = row.reshape(NW, chunks, K)
    col3 = col.reshape(NW, chunks, K)
    w3 = edge_weight.reshape(NW, chunks, K)

    W1p = jnp.concatenate(
        [W1, jnp.zeros((h, cp - c_out), jnp.float32)], axis=1)
    b0r = b0.reshape(1, h)
    b1r = jnp.concatenate(
        [b1, jnp.zeros((cp - c_out,), jnp.float32)]).reshape(1, cp)

    degp = _deg_sc(row3, w3, jnp.zeros((npad,), jnp.float32), npad, chunks)
    degp2 = degp.reshape(NC, npad).T
    g0, dinv = _tc1(x, W0, degp2, n, d, h, 400)
    p1 = _spmm_sc(g0, col3, row3, w3, jnp.zeros((npad, h), jnp.float32),
                  npad, h, chunks)
    g1 = _tc2(p1, g0, dinv, b0r, W1p, n, h, cp, 400)
    p2 = _spmm_sc(g1, col3, row3, w3, jnp.zeros((npad, cp), jnp.float32),
                  npad, cp, chunks)
    out = _tc3(p2, g1, dinv, b1r, n, cp, 400)
    return out[:, :c_out]


# speedup vs baseline: 18.8587x; 1.0297x over previous
import functools

import jax
import jax.numpy as jnp
from jax import lax
from jax.experimental import pallas as pl
from jax.experimental.pallas import tpu as pltpu
from jax.experimental.pallas import tpu_sc as plsc

NC = 2
NS = 16
NW = NC * NS
K = 80


def _sc_mesh():
    return plsc.VectorSubcoreMesh(
        core_axis_name="c", subcore_axis_name="s",
        num_cores=NC, num_subcores=NS)


def _deg_sc(row3, w3, zeros_n, npad, chunks):
    rpt = npad // NS

    @functools.partial(
        pl.kernel,
        out_type=jax.ShapeDtypeStruct((NC * npad,), jnp.float32),
        mesh=_sc_mesh(),
        scratch_types=[
            pltpu.VMEM((chunks, K), jnp.int32),
            pltpu.VMEM((chunks, K), jnp.float32),
            pltpu.VMEM((rpt,), jnp.float32),
            pltpu.VMEM_SHARED((npad,), jnp.float32),
        ],
        compiler_params=pltpu.CompilerParams(use_tc_tiling_on_sc=False, needs_layout_passes=False),
    )
    def deg(row_hbm, w_hbm, z_hbm, out_hbm, row_t, w_t, obuf, acc):
        c = lax.axis_index("c")
        s = lax.axis_index("s")
        wid = s * NC + c
        pltpu.sync_copy(z_hbm.at[pl.ds(s * rpt, rpt)], obuf)
        pltpu.sync_copy(obuf, acc.at[pl.ds(s * rpt, rpt)])
        pltpu.sync_copy(row_hbm.at[wid], row_t)
        pltpu.sync_copy(w_hbm.at[wid], w_t)
        plsc.subcore_barrier()

        def chunk_body(i, carry):
            pltpu.sync_copy(w_t.at[i], acc.at[row_t.at[i]], add=True)
            return carry

        lax.fori_loop(0, chunks, chunk_body, 0)
        plsc.subcore_barrier()
        pltpu.sync_copy(acc.at[pl.ds(s * rpt, rpt)], obuf)
        pltpu.sync_copy(obuf, out_hbm.at[pl.ds(c * npad + s * rpt, rpt)])

    return deg(row3, w3, zeros_n)


NBUF = 5


def _spmm_sc(table, col3, row3, w3, zeros_nf, npad, f, chunks):
    rpt = npad // NS
    assert chunks % NBUF == 0

    @functools.partial(
        pl.kernel,
        out_type=jax.ShapeDtypeStruct((NC, npad, f), jnp.float32),
        mesh=_sc_mesh(),
        scratch_types=[
            pltpu.VMEM((chunks, K), jnp.int32),
            pltpu.VMEM((chunks, K), jnp.int32),
            pltpu.VMEM((chunks, K), jnp.float32),
            [pltpu.VMEM((K, f), jnp.float32) for _ in range(NBUF)],
            pltpu.VMEM((rpt // 4, f), jnp.float32),
            pltpu.VMEM_SHARED((npad, f), jnp.float32),
            [pltpu.SemaphoreType.DMA for _ in range(NBUF)],
        ],
        compiler_params=pltpu.CompilerParams(use_tc_tiling_on_sc=False, needs_layout_passes=False),
    )
    def spmm(table_hbm, col_hbm, row_hbm, w_hbm, z_hbm, out_hbm,
             col_t, row_t, w_t, gbufs, obuf, acc, sems):
        c = lax.axis_index("c")
        s = lax.axis_index("s")
        wid = s * NC + c
        qr = rpt // 4
        pltpu.sync_copy(z_hbm.at[pl.ds(0, qr)], obuf)
        for q in range(4):
            pltpu.sync_copy(obuf, acc.at[pl.ds(s * rpt + q * qr, qr)])
        pltpu.sync_copy(col_hbm.at[wid], col_t)
        pltpu.sync_copy(row_hbm.at[wid], row_t)
        pltpu.sync_copy(w_hbm.at[wid], w_t)
        plsc.subcore_barrier()

        for b in range(NBUF):
            pltpu.async_copy(table_hbm.at[col_t.at[b]], gbufs[b], sems[b])

        def group_body(i, carry):
            for b in range(NBUF):
                ci = i * NBUF + b
                pltpu.make_async_copy(
                    table_hbm.at[col_t.at[ci]], gbufs[b], sems[b]).wait()
                ivec = jnp.full((16,), ci, jnp.int32)
                gb = gbufs[b]
                for e in range(K):
                    wb = plsc.load_gather(
                        w_t, [ivec, jnp.full((16,), e, jnp.int32)])
                    for j in range(f // 16):
                        sl = pl.ds(j * 16, 16)
                        gb[e, sl] = gb[e, sl] * wb
                pltpu.sync_copy(gb, acc.at[row_t.at[ci]], add=True)
                nc = ci + NBUF

                @pl.when(nc < chunks)
                def _():
                    pltpu.async_copy(
                        table_hbm.at[col_t.at[nc]], gbufs[b], sems[b])
            return carry

        lax.fori_loop(0, chunks // NBUF, group_body, 0)
        plsc.subcore_barrier()
        for q in range(4):
            pltpu.sync_copy(acc.at[pl.ds(s * rpt + q * qr, qr)], obuf)
            pltpu.sync_copy(obuf, out_hbm.at[c, pl.ds(s * rpt + q * qr, qr)])

    return spmm(table, col3, row3, w3, zeros_nf)


def _tc1(x, W0, degp, n, d, h, b):
    def body(x_ref, w0_ref, degp_ref, g0_ref, dinv_ref):
        deg = degp_ref[:, 0:1] + degp_ref[:, 1:2] + 1.0
        dinv = lax.rsqrt(deg)
        hh = jnp.dot(x_ref[...], w0_ref[...],
                     preferred_element_type=jnp.float32)
        g0_ref[...] = hh * dinv
        dinv_ref[...] = dinv

    grid = n // b
    return pl.pallas_call(
        body,
        grid=(grid,),
        in_specs=[
            pl.BlockSpec((b, d), lambda i: (i, 0)),
            pl.BlockSpec((d, h), lambda i: (0, 0)),
            pl.BlockSpec((b, NC), lambda i: (i, 0)),
        ],
        out_specs=[
            pl.BlockSpec((b, h), lambda i: (i, 0)),
            pl.BlockSpec((b, 1), lambda i: (i, 0)),
        ],
        out_shape=[
            jax.ShapeDtypeStruct((n, h), jnp.float32),
            jax.ShapeDtypeStruct((n, 1), jnp.float32),
        ],
    )(x, W0, degp)


def _tc2(p1, g0, dinv, b0r, W1p, n, h, cp, b):
    def body(p_ref, g0_ref, dinv_ref, b0_ref, w1_ref, g1_ref):
        ssum = p_ref[0] + p_ref[1] + g0_ref[...]
        a = jnp.maximum(dinv_ref[...] * ssum + b0_ref[...], 0.0)
        g1_ref[...] = jnp.dot(a, w1_ref[...],
                              preferred_element_type=jnp.float32) * dinv_ref[...]

    grid = n // b
    return pl.pallas_call(
        body,
        grid=(grid,),
        in_specs=[
            pl.BlockSpec((NC, b, h), lambda i: (0, i, 0)),
            pl.BlockSpec((b, h), lambda i: (i, 0)),
            pl.BlockSpec((b, 1), lambda i: (i, 0)),
            pl.BlockSpec((1, h), lambda i: (0, 0)),
            pl.BlockSpec((h, cp), lambda i: (0, 0)),
        ],
        out_specs=pl.BlockSpec((b, cp), lambda i: (i, 0)),
        out_shape=jax.ShapeDtypeStruct((n, cp), jnp.float32),
    )(p1, g0, dinv, b0r, W1p)


def _tc3(p2, g1, dinv, b1r, n, cp, b):
    def body(p_ref, g1_ref, dinv_ref, b1_ref, o_ref):
        ssum = p_ref[0] + p_ref[1] + g1_ref[...]
        o_ref[...] = dinv_ref[...] * ssum + b1_ref[...]

    grid = n // b
    return pl.pallas_call(
        body,
        grid=(grid,),
        in_specs=[
            pl.BlockSpec((NC, b, cp), lambda i: (0, i, 0)),
            pl.BlockSpec((b, cp), lambda i: (i, 0)),
            pl.BlockSpec((b, 1), lambda i: (i, 0)),
            pl.BlockSpec((1, cp), lambda i: (0, 0)),
        ],
        out_specs=pl.BlockSpec((b, cp), lambda i: (i, 0)),
        out_shape=jax.ShapeDtypeStruct((n, cp), jnp.float32),
    )(p2, g1, dinv, b1r)


def kernel(x, edge_index, edge_weight, W0, b0, W1, b1):
    n, d = x.shape
    h = W0.shape[1]
    c_out = W1.shape[1]
    cp = 48
    e = edge_index.shape[1]
    chunks = e // (NW * K)
    assert e == NW * chunks * K and n % NS == 0

    npad = 10240
    assert npad % (8 * NS) == 0 and npad >= n

    row = edge_index[0]
    col = edge_index[1]
    row3 = row.reshape(NW, chunks, K)
    col3 = col.reshape(NW, chunks, K)
    w3 = edge_weight.reshape(NW, chunks, K)

    W1p = jnp.concatenate(
        [W1, jnp.zeros((h, cp - c_out), jnp.float32)], axis=1)
    b0r = b0.reshape(1, h)
    b1r = jnp.concatenate(
        [b1, jnp.zeros((cp - c_out,), jnp.float32)]).reshape(1, cp)

    degp = _deg_sc(row3, w3, jnp.zeros((npad,), jnp.float32), npad, chunks)
    degp2 = degp.reshape(NC, npad).T
    g0, dinv = _tc1(x, W0, degp2, n, d, h, 400)
    p1 = _spmm_sc(g0, col3, row3, w3, jnp.zeros((npad, h), jnp.float32),
                  npad, h, chunks)
    g1 = _tc2(p1, g0, dinv, b0r, W1p, n, h, cp, 400)
    p2 = _spmm_sc(g1, col3, row3, w3, jnp.zeros((npad, cp), jnp.float32),
                  npad, cp, chunks)
    out = _tc3(p2, g1, dinv, b1r, n, cp, 400)
    return out[:, :c_out]
